# Initial kernel scaffold; baseline (speedup 1.0000x reference)
import jax
import jax.numpy as jnp
from jax.experimental import pallas as pl


def kernel(pos, node_features, mask_node_features, mask_atom, mask_position, batch, target, W_t0, b_t0, W_f1_0, W_f1_1, W_f1_2, b_f1, W_f2, b_f2, W_p1_0, W_p1_1, W_p1_2, b_p1, W_p2_0, W_p2_1, W_p2_2, b_p2):
    raise NotImplementedError("write your pallas kernel here")


# speedup vs baseline: 2.9240x; 2.9240x over previous
import math

import jax
import jax.numpy as jnp
import numpy as np
from jax.experimental import pallas as pl
from jax.experimental.pallas import tpu as pltpu

N = 50000
G = 512
RES = 32
NUM_GAUSS = 64
TEMP = 0.01

B = 1000
NB = N // B
GB = 128
HIGH = jax.lax.Precision.HIGHEST
DEF = jax.lax.Precision.DEFAULT

_SQ128 = math.sqrt(128.0)
_SQ64 = math.sqrt(64.0)
_SQ32 = math.sqrt(32.0)

_C0 = 0.5 / math.sqrt(math.pi)
_C1 = math.sqrt(3.0 / (4.0 * math.pi))
_C2A = math.sqrt(15.0 / (4.0 * math.pi))
_C20 = math.sqrt(5.0 / (16.0 * math.pi))
_C22 = math.sqrt(15.0 / (16.0 * math.pi))


def _np_sh9(vec):
    x = vec[..., 0]; y = vec[..., 1]; z = vec[..., 2]
    return np.stack([
        _C0 * np.ones_like(x), _C1 * y, _C1 * z, _C1 * x,
        _C2A * x * y, _C2A * y * z, _C20 * (2 * z * z - x * x - y * y),
        _C2A * x * z, _C22 * (x * x - y * y)], axis=-1)


def _np_grid_dirs(res):
    betas = (np.arange(res) + 0.5) / res * np.pi
    alphas = np.arange(res) / res * 2.0 * np.pi
    b, a = np.meshgrid(betas, alphas, indexing='ij')
    x = np.sin(b) * np.cos(a); y = np.sin(b) * np.sin(a); z = np.cos(b)
    return np.stack([x, y, z], -1).reshape(-1, 3).astype(np.float32)


_YT_NP = _np_sh9(_np_grid_dirs(RES)).astype(np.float32).T
_GS_STEP = 10.0 / (NUM_GAUSS - 1)
_GS_COEFF = -0.5 / (_GS_STEP * _GS_STEP)


def _perm_md_to_dm(mul, d):
    p = np.zeros((mul * d, mul * d), np.float32)
    for m in range(mul):
        for j in range(d):
            p[m * d + j, j * mul + m] = 1.0
    return jnp.asarray(p)


_silu = jax.nn.silu
_sigmoid = jax.nn.sigmoid


def _node_body(nf_ref, pos_ref, bcol_ref, brow_ref, mnf0_ref, b1d_ref, b2d_ref,
               tgt_ref, offs_ref, P3_ref, P5_ref,
               Wt0_ref, bt0_ref, W0_ref, W1_ref, W2_ref, bf1_ref, Wf2_ref, bf2_ref,
               gf_out, ctr_out,
               base0_s, base1_s, base2_s, m_s, acc_s):
    i = pl.program_id(0)

    @pl.when(i == 0)
    def _init():
        tgt = tgt_ref[...]
        dmat = tgt - offs_ref[...]
        te = jnp.exp(_GS_COEFF * dmat * dmat)
        t_scal = jnp.dot(te, Wt0_ref[...], precision=DEF) / _SQ64 + bt0_ref[...]
        base0_s[...] = mnf0_ref[...] + _silu(t_scal)
        base1_s[...] = b1d_ref[...]
        base2_s[...] = b2d_ref[...]
        m_s[...] = jnp.full((G, 1), -1e30, jnp.float32)
        acc_s[...] = jnp.zeros((G, 484), jnp.float32)

    bcol = bcol_ref[...]
    brow = brow_ref[0]
    iota_row = jax.lax.broadcasted_iota(jnp.int32, (1, G), 1)
    iota_col = jax.lax.broadcasted_iota(jnp.int32, (G, 1), 0)
    oh = (bcol == iota_row).astype(jnp.float32)
    ohT = (iota_col == brow).astype(jnp.float32)

    nf = nf_ref[...]
    x0 = nf[:, :128] + jnp.dot(oh, base0_s[...], precision=HIGH)
    nf1d = jnp.dot(nf[:, 128:320], P3_ref[...], precision=HIGH)
    nf2d = jnp.dot(nf[:, 320:480], P5_ref[...], precision=HIGH)
    x1 = nf1d + jnp.dot(oh, base1_s[...], precision=HIGH)
    x2 = nf2d + jnp.dot(oh, base2_s[...], precision=HIGH)

    x0 = x0 - jnp.mean(x0, axis=1, keepdims=True)
    x0 = x0 * jax.lax.rsqrt(jnp.mean(x0 * x0, axis=1, keepdims=True) + 1e-5)
    sq1 = x1[:, 0:64] ** 2 + x1[:, 64:128] ** 2 + x1[:, 128:192] ** 2
    x1 = x1 * jax.lax.rsqrt(jnp.mean(sq1, axis=1, keepdims=True) + 1e-5)
    sq2 = (x2[:, 0:32] ** 2 + x2[:, 32:64] ** 2 + x2[:, 64:96] ** 2
           + x2[:, 96:128] ** 2 + x2[:, 128:160] ** 2)
    x2 = x2 * jax.lax.rsqrt(jnp.mean(sq2, axis=1, keepdims=True) + 1e-5)

    o0 = jnp.dot(x0, W0_ref[...], precision=DEF) / _SQ128 + bf1_ref[...]
    o1 = [jnp.dot(x1[:, 64 * d:64 * (d + 1)], W1_ref[...], precision=DEF) / _SQ64
          for d in range(3)]
    o2 = [jnp.dot(x2[:, 32 * d:32 * (d + 1)], W2_ref[...], precision=DEF) / _SQ32
          for d in range(5)]

    s = _silu(o0[:, :128])
    g = _sigmoid(o0[:, 128:224])
    g1 = g[:, :64]
    g2 = g[:, 64:96]
    v1 = [o * g1 for o in o1]
    v2 = [o * g2 for o in o2]

    logit = jnp.dot(s, Wf2_ref[...], precision=DEF) / _SQ128 + bf2_ref[...]

    lmax = jnp.max(jnp.where(ohT > 0, jnp.transpose(logit), -1e30),
                   axis=1, keepdims=True)
    m_old = m_s[...]
    m_new = jnp.maximum(m_old, lmax)
    scale = jnp.exp(m_old - m_new)
    m_s[...] = m_new

    m_g = jnp.dot(oh, m_new, precision=HIGH)
    w = jnp.exp(logit - m_g)

    payload = jnp.concatenate(
        [s * w] + [v * w for v in v1] + [v * w for v in v2]
        + [pos_ref[...] * w, w], axis=1)
    acc_s[...] = acc_s[...] * scale + jnp.dot(ohT, payload, precision=HIGH)

    @pl.when(i == NB - 1)
    def _fin():
        acc = acc_s[...]
        den = acc[:, 483:484]
        inv = jnp.where(den > 0, 1.0 / den, 0.0)
        gf_out[...] = acc[:, :480] * inv
        ctr_out[...] = acc[:, 480:483] * inv


def _head_body(gf_ref, ctr_ref, mpos_ref, W0_ref, W1_ref, W2_ref, bp1_ref,
               Wp20_ref, Wp21_ref, Wp22_ref, bp2_ref, YT_ref,
               res_out, lab_out, len_out):
    gf = gf_ref[...]
    o0 = jnp.dot(gf[:, :128], W0_ref[...], precision=DEF) / _SQ128 + bp1_ref[...]
    o1 = [jnp.dot(gf[:, 128 + 64 * d:128 + 64 * (d + 1)], W1_ref[...],
                  precision=DEF) / _SQ64 for d in range(3)]
    o2 = [jnp.dot(gf[:, 320 + 32 * d:320 + 32 * (d + 1)], W2_ref[...],
                  precision=DEF) / _SQ32 for d in range(5)]

    s = _silu(o0[:, :128])
    g = _sigmoid(o0[:, 128:224])
    h1 = [o * g[:, :64] for o in o1]
    h2 = [o * g[:, 64:96] for o in o2]

    p0 = jnp.dot(s, Wp20_ref[...], precision=DEF) / _SQ128 + bp2_ref[...]
    p1 = [jnp.dot(h, Wp21_ref[...], precision=DEF) / _SQ64 for h in h1]
    p2 = [jnp.dot(h, Wp22_ref[...], precision=DEF) / _SQ32 for h in h2]

    YT = YT_ref[...]

    def gl_c(c):
        cc = jnp.concatenate(
            [p0[:, c:c + 1]] + [p[:, c:c + 1] for p in p1]
            + [p[:, c:c + 1] for p in p2], axis=1)
        return jnp.dot(cc, YT, precision=DEF)

    mx = gl_c(0)
    for c in range(1, 16):
        mx = jnp.maximum(mx, gl_c(c))
    ssum = jnp.zeros_like(mx)
    for c in range(16):
        ssum = ssum + jnp.exp(gl_c(c) - mx)
    plog = jnp.log(ssum) / jnp.float32(TEMP)
    rmax = jnp.max(plog, axis=1, keepdims=True)
    e = jnp.exp(plog - rmax)
    res_out[...] = e / jnp.sum(e, axis=1, keepdims=True)

    lp = mpos_ref[...] - ctr_ref[...]
    x = lp[:, 0:1]; y = lp[:, 1:2]; z = lp[:, 2:3]
    len_out[...] = jnp.sqrt(x * x + y * y + z * z)
    sh = jnp.concatenate([
        _C0 * jnp.ones_like(x), _C1 * y, _C1 * z, _C1 * x,
        _C2A * x * y, _C2A * y * z, _C20 * (2 * z * z - x * x - y * y),
        _C2A * x * z, _C22 * (x * x - y * y)], axis=1)
    t = jnp.dot(sh, YT, precision=HIGH) / jnp.float32(TEMP)
    tm = jnp.max(t, axis=1, keepdims=True)
    el = jnp.exp(t - tm)
    lab_out[...] = el / jnp.sum(el, axis=1, keepdims=True)


def _full_spec(shape):
    return pl.BlockSpec(shape, lambda i: tuple(0 for _ in shape))


def node_pass(pos, node_features, mask_node_features, batch, target,
              W_t0, b_t0, W_f1_0, W_f1_1, W_f1_2, b_f1, W_f2, b_f2):
    f32 = jnp.float32
    batch = batch.astype(jnp.int32)
    bcol = batch.reshape(N, 1)
    brow = batch.reshape(NB, 1, B)
    tgt2d = target.reshape(G, 1).astype(f32)
    offs = jnp.linspace(-5.0, 5.0, NUM_GAUSS).reshape(1, NUM_GAUSS).astype(f32)
    mnf0 = mask_node_features[:, :128]
    b1d = mask_node_features[:, 128:320].reshape(G, 64, 3).transpose(0, 2, 1).reshape(G, 192)
    b2d = mask_node_features[:, 320:480].reshape(G, 32, 5).transpose(0, 2, 1).reshape(G, 160)
    P3 = _perm_md_to_dm(64, 3)
    P5 = _perm_md_to_dm(32, 5)
    full = _full_spec

    gf, ctr = pl.pallas_call(
        _node_body,
        grid=(NB,),
        in_specs=[
            pl.BlockSpec((B, 480), lambda i: (i, 0)),
            pl.BlockSpec((B, 3), lambda i: (i, 0)),
            pl.BlockSpec((B, 1), lambda i: (i, 0)),
            pl.BlockSpec((1, 1, B), lambda i: (i, 0, 0)),
            full((G, 128)), full((G, 192)), full((G, 160)),
            full((G, 1)),
            full((1, NUM_GAUSS)),
            full((192, 192)), full((160, 160)),
            full((64, 128)), full((1, 128)),
            full((128, 224)), full((64, 64)), full((32, 32)),
            full((1, 224)), full((128, 1)), full((1, 1)),
        ],
        out_specs=[full((G, 480)), full((G, 3))],
        out_shape=[jax.ShapeDtypeStruct((G, 480), f32),
                   jax.ShapeDtypeStruct((G, 3), f32)],
        scratch_shapes=[
            pltpu.VMEM((G, 128), f32), pltpu.VMEM((G, 192), f32),
            pltpu.VMEM((G, 160), f32),
            pltpu.VMEM((G, 1), f32), pltpu.VMEM((G, 484), f32),
        ],
    )(node_features, pos, bcol, brow, mnf0, b1d, b2d, tgt2d, offs, P3, P5,
      W_t0, b_t0.reshape(1, 128), W_f1_0, W_f1_1, W_f1_2, b_f1.reshape(1, 224),
      W_f2, b_f2.reshape(1, 1))
    return gf, ctr


def head_pass(gf, ctr, mask_position,
              W_p1_0, W_p1_1, W_p1_2, b_p1, W_p2_0, W_p2_1, W_p2_2, b_p2):
    f32 = jnp.float32
    YT = jnp.asarray(_YT_NP)
    full = _full_spec

    res, lab, ln = pl.pallas_call(
        _head_body,
        grid=(G // GB,),
        in_specs=[
            pl.BlockSpec((GB, 480), lambda i: (i, 0)),
            pl.BlockSpec((GB, 3), lambda i: (i, 0)),
            pl.BlockSpec((GB, 3), lambda i: (i, 0)),
            full((128, 224)), full((64, 64)), full((32, 32)), full((1, 224)),
            full((128, 16)), full((64, 16)), full((32, 16)), full((1, 16)),
            full((9, 1024)),
        ],
        out_specs=[pl.BlockSpec((GB, 1024), lambda i: (i, 0)),
                   pl.BlockSpec((GB, 1024), lambda i: (i, 0)),
                   pl.BlockSpec((GB, 1), lambda i: (i, 0))],
        out_shape=[jax.ShapeDtypeStruct((G, 1024), f32),
                   jax.ShapeDtypeStruct((G, 1024), f32),
                   jax.ShapeDtypeStruct((G, 1), f32)],
    )(gf, ctr, mask_position, W_p1_0, W_p1_1, W_p1_2, b_p1.reshape(1, 224),
      W_p2_0, W_p2_1, W_p2_2, b_p2.reshape(1, 16), YT)
    return res, lab, ln


def kernel(pos, node_features, mask_node_features, mask_atom, mask_position,
           batch, target,
           W_t0, b_t0, W_f1_0, W_f1_1, W_f1_2, b_f1, W_f2, b_f2,
           W_p1_0, W_p1_1, W_p1_2, b_p1, W_p2_0, W_p2_1, W_p2_2, b_p2):
    del mask_atom
    gf, ctr = node_pass(pos, node_features, mask_node_features, batch, target,
                        W_t0, b_t0, W_f1_0, W_f1_1, W_f1_2, b_f1, W_f2, b_f2)
    return head_pass(gf, ctr, mask_position,
                     W_p1_0, W_p1_1, W_p1_2, b_p1, W_p2_0, W_p2_1, W_p2_2, b_p2)


# speedup vs baseline: 2.9352x; 1.0038x over previous
import math

import jax
import jax.numpy as jnp
import numpy as np
from jax.experimental import pallas as pl
from jax.experimental.pallas import tpu as pltpu

N = 50000
G = 512
RES = 32
NUM_GAUSS = 64
TEMP = 0.01

B = 2000
NB = N // B
GB = 128
HIGH = jax.lax.Precision.HIGHEST
DEF = jax.lax.Precision.DEFAULT

_SQ128 = math.sqrt(128.0)
_SQ64 = math.sqrt(64.0)
_SQ32 = math.sqrt(32.0)

_C0 = 0.5 / math.sqrt(math.pi)
_C1 = math.sqrt(3.0 / (4.0 * math.pi))
_C2A = math.sqrt(15.0 / (4.0 * math.pi))
_C20 = math.sqrt(5.0 / (16.0 * math.pi))
_C22 = math.sqrt(15.0 / (16.0 * math.pi))


def _np_sh9(vec):
    x = vec[..., 0]; y = vec[..., 1]; z = vec[..., 2]
    return np.stack([
        _C0 * np.ones_like(x), _C1 * y, _C1 * z, _C1 * x,
        _C2A * x * y, _C2A * y * z, _C20 * (2 * z * z - x * x - y * y),
        _C2A * x * z, _C22 * (x * x - y * y)], axis=-1)


def _np_grid_dirs(res):
    betas = (np.arange(res) + 0.5) / res * np.pi
    alphas = np.arange(res) / res * 2.0 * np.pi
    b, a = np.meshgrid(betas, alphas, indexing='ij')
    x = np.sin(b) * np.cos(a); y = np.sin(b) * np.sin(a); z = np.cos(b)
    return np.stack([x, y, z], -1).reshape(-1, 3).astype(np.float32)


_YT_NP = _np_sh9(_np_grid_dirs(RES)).astype(np.float32).T
_GS_STEP = 10.0 / (NUM_GAUSS - 1)
_GS_COEFF = -0.5 / (_GS_STEP * _GS_STEP)


def _perm_md_to_dm(mul, d):
    p = np.zeros((mul * d, mul * d), np.float32)
    for m in range(mul):
        for j in range(d):
            p[m * d + j, j * mul + m] = 1.0
    return jnp.asarray(p)


_silu = jax.nn.silu
_sigmoid = jax.nn.sigmoid


def _node_body(nf_ref, pos_ref, bcol_ref, brow_ref, mnf0_ref, b1d_ref, b2d_ref,
               tgt_ref, offs_ref, P3_ref, P5_ref,
               Wt0_ref, bt0_ref, W0_ref, W1_ref, W2_ref, bf1_ref, Wf2_ref, bf2_ref,
               gf_out, ctr_out,
               base0_s, base1_s, base2_s, m_s, acc_s):
    i = pl.program_id(0)

    @pl.when(i == 0)
    def _init():
        tgt = tgt_ref[...]
        dmat = tgt - offs_ref[...]
        te = jnp.exp(_GS_COEFF * dmat * dmat)
        t_scal = jnp.dot(te, Wt0_ref[...], precision=DEF) / _SQ64 + bt0_ref[...]
        base0_s[...] = mnf0_ref[...] + _silu(t_scal)
        base1_s[...] = b1d_ref[...]
        base2_s[...] = b2d_ref[...]
        m_s[...] = jnp.full((G, 1), -1e30, jnp.float32)
        acc_s[...] = jnp.zeros((G, 484), jnp.float32)

    bcol = bcol_ref[...]
    brow = brow_ref[0]
    iota_row = jax.lax.broadcasted_iota(jnp.int32, (1, G), 1)
    iota_col = jax.lax.broadcasted_iota(jnp.int32, (G, 1), 0)
    oh = (bcol == iota_row).astype(jnp.float32)
    ohT = (iota_col == brow).astype(jnp.float32)

    nf = nf_ref[...]
    x0 = nf[:, :128] + jnp.dot(oh, base0_s[...], precision=HIGH)
    nf1d = jnp.dot(nf[:, 128:320], P3_ref[...], precision=HIGH)
    nf2d = jnp.dot(nf[:, 320:480], P5_ref[...], precision=HIGH)
    x1 = nf1d + jnp.dot(oh, base1_s[...], precision=HIGH)
    x2 = nf2d + jnp.dot(oh, base2_s[...], precision=HIGH)

    x0 = x0 - jnp.mean(x0, axis=1, keepdims=True)
    x0 = x0 * jax.lax.rsqrt(jnp.mean(x0 * x0, axis=1, keepdims=True) + 1e-5)
    sq1 = x1[:, 0:64] ** 2 + x1[:, 64:128] ** 2 + x1[:, 128:192] ** 2
    x1 = x1 * jax.lax.rsqrt(jnp.mean(sq1, axis=1, keepdims=True) + 1e-5)
    sq2 = (x2[:, 0:32] ** 2 + x2[:, 32:64] ** 2 + x2[:, 64:96] ** 2
           + x2[:, 96:128] ** 2 + x2[:, 128:160] ** 2)
    x2 = x2 * jax.lax.rsqrt(jnp.mean(sq2, axis=1, keepdims=True) + 1e-5)

    o0 = jnp.dot(x0, W0_ref[...], precision=DEF) / _SQ128 + bf1_ref[...]
    o1 = [jnp.dot(x1[:, 64 * d:64 * (d + 1)], W1_ref[...], precision=DEF) / _SQ64
          for d in range(3)]
    o2 = [jnp.dot(x2[:, 32 * d:32 * (d + 1)], W2_ref[...], precision=DEF) / _SQ32
          for d in range(5)]

    s = _silu(o0[:, :128])
    g = _sigmoid(o0[:, 128:224])
    g1 = g[:, :64]
    g2 = g[:, 64:96]
    v1 = [o * g1 for o in o1]
    v2 = [o * g2 for o in o2]

    logit = jnp.dot(s, Wf2_ref[...], precision=DEF) / _SQ128 + bf2_ref[...]

    lmax = jnp.max(jnp.where(ohT > 0, jnp.transpose(logit), -1e30),
                   axis=1, keepdims=True)
    m_old = m_s[...]
    m_new = jnp.maximum(m_old, lmax)
    scale = jnp.exp(m_old - m_new)
    m_s[...] = m_new

    m_g = jnp.dot(oh, m_new, precision=HIGH)
    w = jnp.exp(logit - m_g)

    payload = jnp.concatenate(
        [s * w] + [v * w for v in v1] + [v * w for v in v2]
        + [pos_ref[...] * w, w], axis=1)
    acc_s[...] = acc_s[...] * scale + jnp.dot(ohT, payload, precision=HIGH)

    @pl.when(i == NB - 1)
    def _fin():
        acc = acc_s[...]
        den = acc[:, 483:484]
        inv = jnp.where(den > 0, 1.0 / den, 0.0)
        gf_out[...] = acc[:, :480] * inv
        ctr_out[...] = acc[:, 480:483] * inv


def _head_body(gf_ref, ctr_ref, mpos_ref, W0_ref, W1_ref, W2_ref, bp1_ref,
               Wp20_ref, Wp21_ref, Wp22_ref, bp2_ref, YT_ref,
               res_out, lab_out, len_out):
    gf = gf_ref[...]
    o0 = jnp.dot(gf[:, :128], W0_ref[...], precision=DEF) / _SQ128 + bp1_ref[...]
    o1 = [jnp.dot(gf[:, 128 + 64 * d:128 + 64 * (d + 1)], W1_ref[...],
                  precision=DEF) / _SQ64 for d in range(3)]
    o2 = [jnp.dot(gf[:, 320 + 32 * d:320 + 32 * (d + 1)], W2_ref[...],
                  precision=DEF) / _SQ32 for d in range(5)]

    s = _silu(o0[:, :128])
    g = _sigmoid(o0[:, 128:224])
    h1 = [o * g[:, :64] for o in o1]
    h2 = [o * g[:, 64:96] for o in o2]

    p0 = jnp.dot(s, Wp20_ref[...], precision=DEF) / _SQ128 + bp2_ref[...]
    p1 = [jnp.dot(h, Wp21_ref[...], precision=DEF) / _SQ64 for h in h1]
    p2 = [jnp.dot(h, Wp22_ref[...], precision=DEF) / _SQ32 for h in h2]

    YT = YT_ref[...]

    def gl_c(c):
        cc = jnp.concatenate(
            [p0[:, c:c + 1]] + [p[:, c:c + 1] for p in p1]
            + [p[:, c:c + 1] for p in p2], axis=1)
        return jnp.dot(cc, YT, precision=DEF)

    mx = gl_c(0)
    for c in range(1, 16):
        mx = jnp.maximum(mx, gl_c(c))
    ssum = jnp.zeros_like(mx)
    for c in range(16):
        ssum = ssum + jnp.exp(gl_c(c) - mx)
    plog = jnp.log(ssum) / jnp.float32(TEMP)
    rmax = jnp.max(plog, axis=1, keepdims=True)
    e = jnp.exp(plog - rmax)
    res_out[...] = e / jnp.sum(e, axis=1, keepdims=True)

    lp = mpos_ref[...] - ctr_ref[...]
    x = lp[:, 0:1]; y = lp[:, 1:2]; z = lp[:, 2:3]
    len_out[...] = jnp.sqrt(x * x + y * y + z * z)
    sh = jnp.concatenate([
        _C0 * jnp.ones_like(x), _C1 * y, _C1 * z, _C1 * x,
        _C2A * x * y, _C2A * y * z, _C20 * (2 * z * z - x * x - y * y),
        _C2A * x * z, _C22 * (x * x - y * y)], axis=1)
    t = jnp.dot(sh, YT, precision=HIGH) / jnp.float32(TEMP)
    tm = jnp.max(t, axis=1, keepdims=True)
    el = jnp.exp(t - tm)
    lab_out[...] = el / jnp.sum(el, axis=1, keepdims=True)


def _full_spec(shape):
    return pl.BlockSpec(shape, lambda i: tuple(0 for _ in shape))


def node_pass(pos, node_features, mask_node_features, batch, target,
              W_t0, b_t0, W_f1_0, W_f1_1, W_f1_2, b_f1, W_f2, b_f2):
    f32 = jnp.float32
    batch = batch.astype(jnp.int32)
    bcol = batch.reshape(N, 1)
    brow = batch.reshape(NB, 1, B)
    tgt2d = target.reshape(G, 1).astype(f32)
    offs = jnp.linspace(-5.0, 5.0, NUM_GAUSS).reshape(1, NUM_GAUSS).astype(f32)
    mnf0 = mask_node_features[:, :128]
    b1d = mask_node_features[:, 128:320].reshape(G, 64, 3).transpose(0, 2, 1).reshape(G, 192)
    b2d = mask_node_features[:, 320:480].reshape(G, 32, 5).transpose(0, 2, 1).reshape(G, 160)
    P3 = _perm_md_to_dm(64, 3)
    P5 = _perm_md_to_dm(32, 5)
    full = _full_spec

    gf, ctr = pl.pallas_call(
        _node_body,
        grid=(NB,),
        in_specs=[
            pl.BlockSpec((B, 480), lambda i: (i, 0)),
            pl.BlockSpec((B, 3), lambda i: (i, 0)),
            pl.BlockSpec((B, 1), lambda i: (i, 0)),
            pl.BlockSpec((1, 1, B), lambda i: (i, 0, 0)),
            full((G, 128)), full((G, 192)), full((G, 160)),
            full((G, 1)),
            full((1, NUM_GAUSS)),
            full((192, 192)), full((160, 160)),
            full((64, 128)), full((1, 128)),
            full((128, 224)), full((64, 64)), full((32, 32)),
            full((1, 224)), full((128, 1)), full((1, 1)),
        ],
        out_specs=[full((G, 480)), full((G, 3))],
        out_shape=[jax.ShapeDtypeStruct((G, 480), f32),
                   jax.ShapeDtypeStruct((G, 3), f32)],
        scratch_shapes=[
            pltpu.VMEM((G, 128), f32), pltpu.VMEM((G, 192), f32),
            pltpu.VMEM((G, 160), f32),
            pltpu.VMEM((G, 1), f32), pltpu.VMEM((G, 484), f32),
        ],
    )(node_features, pos, bcol, brow, mnf0, b1d, b2d, tgt2d, offs, P3, P5,
      W_t0, b_t0.reshape(1, 128), W_f1_0, W_f1_1, W_f1_2, b_f1.reshape(1, 224),
      W_f2, b_f2.reshape(1, 1))
    return gf, ctr


def head_pass(gf, ctr, mask_position,
              W_p1_0, W_p1_1, W_p1_2, b_p1, W_p2_0, W_p2_1, W_p2_2, b_p2):
    f32 = jnp.float32
    YT = jnp.asarray(_YT_NP)
    full = _full_spec

    res, lab, ln = pl.pallas_call(
        _head_body,
        grid=(G // GB,),
        in_specs=[
            pl.BlockSpec((GB, 480), lambda i: (i, 0)),
            pl.BlockSpec((GB, 3), lambda i: (i, 0)),
            pl.BlockSpec((GB, 3), lambda i: (i, 0)),
            full((128, 224)), full((64, 64)), full((32, 32)), full((1, 224)),
            full((128, 16)), full((64, 16)), full((32, 16)), full((1, 16)),
            full((9, 1024)),
        ],
        out_specs=[pl.BlockSpec((GB, 1024), lambda i: (i, 0)),
                   pl.BlockSpec((GB, 1024), lambda i: (i, 0)),
                   pl.BlockSpec((GB, 1), lambda i: (i, 0))],
        out_shape=[jax.ShapeDtypeStruct((G, 1024), f32),
                   jax.ShapeDtypeStruct((G, 1024), f32),
                   jax.ShapeDtypeStruct((G, 1), f32)],
    )(gf, ctr, mask_position, W_p1_0, W_p1_1, W_p1_2, b_p1.reshape(1, 224),
      W_p2_0, W_p2_1, W_p2_2, b_p2.reshape(1, 16), YT)
    return res, lab, ln


def kernel(pos, node_features, mask_node_features, mask_atom, mask_position,
           batch, target,
           W_t0, b_t0, W_f1_0, W_f1_1, W_f1_2, b_f1, W_f2, b_f2,
           W_p1_0, W_p1_1, W_p1_2, b_p1, W_p2_0, W_p2_1, W_p2_2, b_p2):
    del mask_atom
    gf, ctr = node_pass(pos, node_features, mask_node_features, batch, target,
                        W_t0, b_t0, W_f1_0, W_f1_1, W_f1_2, b_f1, W_f2, b_f2)
    return head_pass(gf, ctr, mask_position,
                     W_p1_0, W_p1_1, W_p1_2, b_p1, W_p2_0, W_p2_1, W_p2_2, b_p2)
